# baseline (speedup 1.0000x reference)
import functools

import jax
import jax.numpy as jnp
from jax import lax
from jax.experimental import pallas as pl
from jax.experimental.pallas import tpu as pltpu
from jax.experimental.pallas import tpu_sc as plsc

N_NODES = 10000
N_EDGES = 320000
IN_CH = 128
HID = 64
HID2 = 32
OUT_CH = 1

NUM_SC = 2
NUM_TILES = 16
NW = NUM_SC * NUM_TILES
CHUNK = 128
CHUNKS_PER_TILE = 80
EDGES_PER_TILE = CHUNK * CHUNKS_PER_TILE
E_PAD = EDGES_PER_TILE * NW
TOT_CHUNKS = E_PAD // CHUNK
CH_C0 = 32
CH_C1 = 128
DUMMY_ROW = N_NODES
ACC_ROWS = 10112
STRIPE = ACC_ROWS // NUM_TILES
LAST_STRIPE = N_NODES - (NUM_TILES - 1) * STRIPE

_MESH = plsc.VectorSubcoreMesh(core_axis_name="c", subcore_axis_name="s")
_SC_PARAMS = pltpu.CompilerParams(use_tc_tiling_on_sc=False)
_MM = (((1,), (0,)), ((), ()))


def _zero_fill(buf, nrows, width):
    @pl.loop(0, nrows)
    def _(i):
        @pl.loop(0, width // 16)
        def _(k):
            buf[i, pl.ds(k * 16, 16)] = jnp.zeros((16,), jnp.float32)


def _make_sc_agg(width, nbuf, ch0, ch1):
    out_t = [jax.ShapeDtypeStruct((N_NODES, width), jnp.float32)] * 2
    assert ch0 % nbuf == 0 and ch1 % nbuf == 0
    assert (ch0 + ch1) * NUM_TILES == TOT_CHUNKS
    chmax = max(ch0, ch1)

    @functools.partial(
        pl.kernel,
        out_type=out_t,
        mesh=_MESH,
        compiler_params=_SC_PARAMS,
        scratch_types=[
            pltpu.VMEM((chmax, CHUNK), jnp.int32),
            pltpu.VMEM((chmax, CHUNK), jnp.int32),
            pltpu.VMEM((nbuf, CHUNK, width), jnp.float32),
            pltpu.VMEM_SHARED((ACC_ROWS, width), jnp.float32),
            pltpu.SemaphoreType.DMA((nbuf,)),
            pltpu.SemaphoreType.DMA((nbuf,)),
        ],
    )
    def agg(table_hbm, src0_hbm, dst0_hbm, src1_hbm, dst1_hbm, out0, out1,
            src_v, dst_v, rows_v, acc, gsem, ssem):
        c = lax.axis_index("c")
        s = lax.axis_index("s")
        _zero_fill(rows_v.at[0], CHUNK, width)
        zbase = s * STRIPE
        for off in range(0, STRIPE, CHUNK):
            nrow = min(CHUNK, STRIPE - off)
            pltpu.sync_copy(rows_v.at[0].at[pl.ds(0, nrow)],
                            acc.at[pl.ds(zbase + off, nrow)])
        plsc.subcore_barrier()

        def run(ch, src_h, dst_h):
            pltpu.sync_copy(src_h.at[s], src_v.at[pl.ds(0, ch)])
            pltpu.sync_copy(dst_h.at[s], dst_v.at[pl.ds(0, ch)])

            @pl.loop(0, ch, step=nbuf)
            def _(j):
                gds = [pltpu.async_copy(table_hbm.at[src_v.at[j + b]],
                                        rows_v.at[b], gsem.at[b])
                       for b in range(nbuf)]
                sds = []
                for b in range(nbuf):
                    gds[b].wait()
                    sds.append(pltpu.async_copy(rows_v.at[b],
                                                acc.at[dst_v.at[j + b]],
                                                ssem.at[b], add=True))
                for b in range(nbuf):
                    sds[b].wait()

        @pl.when(c == 0)
        def _():
            run(ch0, src0_hbm, dst0_hbm)

        @pl.when(c == 1)
        def _():
            run(ch1, src1_hbm, dst1_hbm)

        plsc.subcore_barrier()
        _copy_out(acc, out0, out1, c, s)

    return agg


def _copy_out(acc, out0, out1, c, s):
    ob = s * STRIPE

    def stripe_to(dst):
        @pl.when(s < NUM_TILES - 1)
        def _():
            pltpu.sync_copy(acc.at[pl.ds(ob, STRIPE)],
                            dst.at[pl.ds(ob, STRIPE)])

        @pl.when(s == NUM_TILES - 1)
        def _():
            pltpu.sync_copy(acc.at[pl.ds(ob, LAST_STRIPE)],
                            dst.at[pl.ds(ob, LAST_STRIPE)])

    @pl.when(c == 0)
    def _():
        stripe_to(out0)

    @pl.when(c == 1)
    def _():
        stripe_to(out1)


_CNT_W = 16
_CNT_NBUF = 8


def _sc_count(dst_hbm_arr):
    out_t = [jax.ShapeDtypeStruct((N_NODES, _CNT_W), jnp.float32)] * 2

    @functools.partial(
        pl.kernel,
        out_type=out_t,
        mesh=_MESH,
        compiler_params=_SC_PARAMS,
        scratch_types=[
            pltpu.VMEM((CHUNKS_PER_TILE, CHUNK), jnp.int32),
            pltpu.VMEM((CHUNK, _CNT_W), jnp.float32),
            pltpu.VMEM_SHARED((ACC_ROWS, _CNT_W), jnp.float32),
            pltpu.SemaphoreType.DMA,
        ],
    )
    def cnt(dst_hbm, out0, out1, dst_v, ones_v, acc, sem):
        c = lax.axis_index("c")
        s = lax.axis_index("s")
        w = c * NUM_TILES + s
        pltpu.sync_copy(dst_hbm.at[w], dst_v)
        _zero_fill(ones_v, CHUNK, _CNT_W)
        zbase = s * STRIPE
        for off in range(0, STRIPE, CHUNK):
            nrow = min(CHUNK, STRIPE - off)
            pltpu.sync_copy(ones_v.at[pl.ds(0, nrow)],
                            acc.at[pl.ds(zbase + off, nrow)])

        @pl.loop(0, CHUNK)
        def _(i):
            ones_v[i, pl.ds(0, _CNT_W)] = jnp.ones((_CNT_W,), jnp.float32)

        plsc.subcore_barrier()

        @pl.loop(0, CHUNKS_PER_TILE, step=_CNT_NBUF)
        def _(j):
            sds = [pltpu.async_copy(ones_v, acc.at[dst_v.at[j + b]],
                                    sem, add=True)
                   for b in range(_CNT_NBUF)]
            for d in sds:
                d.wait()

        plsc.subcore_barrier()
        _copy_out(acc, out0, out1, c, s)

    return cnt(dst_hbm_arr)


_BLK = 2000


def _tc_layer1(a0a, a1a, a0b, a1b, x, c0, c1, wl, wr, b):
    def body(a0a_ref, a1a_ref, a0b_ref, a1b_ref, x_ref, c0_ref, c1_ref,
             wl_ref, wr_ref, b_ref, h_ref):
        cntv = jnp.maximum(c0_ref[:, 0:1] + c1_ref[:, 0:1], 1.0)
        agg = jnp.concatenate([a0a_ref[...] + a1a_ref[...],
                               a0b_ref[...] + a1b_ref[...]], axis=1)
        mean1 = agg / cntv
        h = (lax.dot_general(mean1, wl_ref[...], _MM) + b_ref[...]
             + lax.dot_general(x_ref[...], wr_ref[...], _MM))
        h_ref[...] = jnp.maximum(h, 0.0)

    return pl.pallas_call(
        body,
        grid=(N_NODES // _BLK,),
        in_specs=[
            pl.BlockSpec((_BLK, HID), lambda i: (i, 0)),
            pl.BlockSpec((_BLK, HID), lambda i: (i, 0)),
            pl.BlockSpec((_BLK, HID), lambda i: (i, 0)),
            pl.BlockSpec((_BLK, HID), lambda i: (i, 0)),
            pl.BlockSpec((_BLK, IN_CH), lambda i: (i, 0)),
            pl.BlockSpec((_BLK, _CNT_W), lambda i: (i, 0)),
            pl.BlockSpec((_BLK, _CNT_W), lambda i: (i, 0)),
            pl.BlockSpec((IN_CH, HID), lambda i: (0, 0)),
            pl.BlockSpec((IN_CH, HID), lambda i: (0, 0)),
            pl.BlockSpec((1, HID), lambda i: (0, 0)),
        ],
        out_specs=pl.BlockSpec((_BLK, HID), lambda i: (i, 0)),
        out_shape=jax.ShapeDtypeStruct((N_NODES, HID), jnp.float32),
    )(a0a, a1a, a0b, a1b, x, c0, c1, wl, wr, b)


def _tc_layer2(b0, b1, h1, c0, c1, wl, wr, b, wc, bc):
    def body(b0_ref, b1_ref, h1_ref, c0_ref, c1_ref, wl_ref, wr_ref, b_ref,
             wc_ref, bc_ref, o_ref):
        cntv = jnp.maximum(c0_ref[:, 0:1] + c1_ref[:, 0:1], 1.0)
        mean2 = (b0_ref[...] + b1_ref[...]) / cntv
        h2 = (lax.dot_general(mean2, wl_ref[...], _MM) + b_ref[...]
              + lax.dot_general(h1_ref[...], wr_ref[...], _MM))
        h2 = jnp.maximum(h2, 0.0)
        o_ref[...] = lax.dot_general(h2, wc_ref[...], _MM) + bc_ref[...]

    return pl.pallas_call(
        body,
        grid=(N_NODES // _BLK,),
        in_specs=[
            pl.BlockSpec((_BLK, HID), lambda i: (i, 0)),
            pl.BlockSpec((_BLK, HID), lambda i: (i, 0)),
            pl.BlockSpec((_BLK, HID), lambda i: (i, 0)),
            pl.BlockSpec((_BLK, _CNT_W), lambda i: (i, 0)),
            pl.BlockSpec((_BLK, _CNT_W), lambda i: (i, 0)),
            pl.BlockSpec((HID, HID2), lambda i: (0, 0)),
            pl.BlockSpec((HID, HID2), lambda i: (0, 0)),
            pl.BlockSpec((1, HID2), lambda i: (0, 0)),
            pl.BlockSpec((HID2, OUT_CH), lambda i: (0, 0)),
            pl.BlockSpec((1, OUT_CH), lambda i: (0, 0)),
        ],
        out_specs=pl.BlockSpec((_BLK, OUT_CH), lambda i: (i, 0)),
        out_shape=jax.ShapeDtypeStruct((N_NODES, OUT_CH), jnp.float32),
    )(b0, b1, h1, c0, c1, wl, wr, b, wc, bc)


def kernel(x, edge_index, Wl1, bl1, Wr1, Wl2, bl2, Wr2, Wc, bc):
    ei = edge_index.astype(jnp.int32)
    pad = E_PAD - N_EDGES
    src = jnp.concatenate([ei[0], jnp.zeros((pad,), jnp.int32)])
    dst = jnp.concatenate([ei[1], jnp.full((pad,), DUMMY_ROW, jnp.int32)])
    src = src.reshape(TOT_CHUNKS, CHUNK)
    dst = dst.reshape(TOT_CHUNKS, CHUNK)
    n0 = NUM_TILES * CH_C0
    src0 = src[:n0].reshape(NUM_TILES, CH_C0, CHUNK)
    dst0 = dst[:n0].reshape(NUM_TILES, CH_C0, CHUNK)
    src1 = src[n0:].reshape(NUM_TILES, CH_C1, CHUNK)
    dst1 = dst[n0:].reshape(NUM_TILES, CH_C1, CHUNK)
    idx = (src0, dst0, src1, dst1)

    cnt0, cnt1 = _sc_count(dst.reshape(NW, CHUNKS_PER_TILE, CHUNK))
    agg64 = _make_sc_agg(HID, 4, CH_C0, CH_C1)
    a0a, a1a = agg64(x[:, :HID], *idx)
    a0b, a1b = agg64(x[:, HID:], *idx)
    h1 = _tc_layer1(a0a, a1a, a0b, a1b, x, cnt0, cnt1, Wl1.T, Wr1.T,
                    bl1.reshape(1, HID))
    b0, b1 = agg64(h1, *idx)
    return _tc_layer2(b0, b1, h1, cnt0, cnt1, Wl2.T, Wr2.T,
                      bl2.reshape(1, HID2), Wc.T, bc.reshape(1, OUT_CH))


# speedup vs baseline: 1.2449x; 1.2449x over previous
import functools

import jax
import jax.numpy as jnp
from jax import lax
from jax.experimental import pallas as pl
from jax.experimental.pallas import tpu as pltpu
from jax.experimental.pallas import tpu_sc as plsc

N_NODES = 10000
N_EDGES = 320000
IN_CH = 128
HID = 64
HID2 = 32
OUT_CH = 1

NUM_SC = 2
NUM_TILES = 16
NW = NUM_SC * NUM_TILES
CHUNK = 128
CHUNKS_PER_TILE = 80
EDGES_PER_TILE = CHUNK * CHUNKS_PER_TILE
E_PAD = EDGES_PER_TILE * NW
TOT_CHUNKS = E_PAD // CHUNK
CH_C0 = 128
CH_C1 = 32
DUMMY_ROW = N_NODES
ACC_ROWS = 10112
STRIPE = ACC_ROWS // NUM_TILES
LAST_STRIPE = N_NODES - (NUM_TILES - 1) * STRIPE

_MESH = plsc.VectorSubcoreMesh(core_axis_name="c", subcore_axis_name="s")
_SC_PARAMS = pltpu.CompilerParams(use_tc_tiling_on_sc=False)
_MM = (((1,), (0,)), ((), ()))


def _zero_fill(buf, nrows, width):
    @pl.loop(0, nrows)
    def _(i):
        @pl.loop(0, width // 16)
        def _(k):
            buf[i, pl.ds(k * 16, 16)] = jnp.zeros((16,), jnp.float32)


def _make_sc_agg(width, nbuf, ch0, ch1):
    out_t = [jax.ShapeDtypeStruct((N_NODES, width), jnp.float32)] * 2
    assert ch0 % nbuf == 0 and ch1 % nbuf == 0
    assert (ch0 + ch1) * NUM_TILES == TOT_CHUNKS
    chmax = max(ch0, ch1)

    @functools.partial(
        pl.kernel,
        out_type=out_t,
        mesh=_MESH,
        compiler_params=_SC_PARAMS,
        scratch_types=[
            pltpu.VMEM((chmax, CHUNK), jnp.int32),
            pltpu.VMEM((chmax, CHUNK), jnp.int32),
            pltpu.VMEM((nbuf, CHUNK, width), jnp.float32),
            pltpu.VMEM_SHARED((ACC_ROWS, width), jnp.float32),
            pltpu.SemaphoreType.DMA((nbuf,)),
            pltpu.SemaphoreType.DMA((nbuf,)),
        ],
    )
    def agg(table_hbm, src0_hbm, dst0_hbm, src1_hbm, dst1_hbm, out0, out1,
            src_v, dst_v, rows_v, acc, gsem, ssem):
        c = lax.axis_index("c")
        s = lax.axis_index("s")
        _zero_fill(rows_v.at[0], CHUNK, width)
        zbase = s * STRIPE
        for off in range(0, STRIPE, CHUNK):
            nrow = min(CHUNK, STRIPE - off)
            pltpu.sync_copy(rows_v.at[0].at[pl.ds(0, nrow)],
                            acc.at[pl.ds(zbase + off, nrow)])
        plsc.subcore_barrier()

        def run(ch, src_h, dst_h):
            pltpu.sync_copy(src_h.at[s], src_v.at[pl.ds(0, ch)])
            pltpu.sync_copy(dst_h.at[s], dst_v.at[pl.ds(0, ch)])

            @pl.loop(0, ch, step=nbuf)
            def _(j):
                gds = [pltpu.async_copy(table_hbm.at[src_v.at[j + b]],
                                        rows_v.at[b], gsem.at[b])
                       for b in range(nbuf)]
                sds = []
                for b in range(nbuf):
                    gds[b].wait()
                    sds.append(pltpu.async_copy(rows_v.at[b],
                                                acc.at[dst_v.at[j + b]],
                                                ssem.at[b], add=True))
                for b in range(nbuf):
                    sds[b].wait()

        @pl.when(c == 0)
        def _():
            run(ch0, src0_hbm, dst0_hbm)

        @pl.when(c == 1)
        def _():
            run(ch1, src1_hbm, dst1_hbm)

        plsc.subcore_barrier()
        _copy_out(acc, out0, out1, c, s)

    return agg


def _copy_out(acc, out0, out1, c, s):
    ob = s * STRIPE

    def stripe_to(dst):
        @pl.when(s < NUM_TILES - 1)
        def _():
            pltpu.sync_copy(acc.at[pl.ds(ob, STRIPE)],
                            dst.at[pl.ds(ob, STRIPE)])

        @pl.when(s == NUM_TILES - 1)
        def _():
            pltpu.sync_copy(acc.at[pl.ds(ob, LAST_STRIPE)],
                            dst.at[pl.ds(ob, LAST_STRIPE)])

    @pl.when(c == 0)
    def _():
        stripe_to(out0)

    @pl.when(c == 1)
    def _():
        stripe_to(out1)


_CNT_W = 16
_CNT_NBUF = 8


def _sc_count(dst_hbm_arr):
    out_t = [jax.ShapeDtypeStruct((N_NODES, _CNT_W), jnp.float32)] * 2

    @functools.partial(
        pl.kernel,
        out_type=out_t,
        mesh=_MESH,
        compiler_params=_SC_PARAMS,
        scratch_types=[
            pltpu.VMEM((CHUNKS_PER_TILE, CHUNK), jnp.int32),
            pltpu.VMEM((CHUNK, _CNT_W), jnp.float32),
            pltpu.VMEM_SHARED((ACC_ROWS, _CNT_W), jnp.float32),
            pltpu.SemaphoreType.DMA,
        ],
    )
    def cnt(dst_hbm, out0, out1, dst_v, ones_v, acc, sem):
        c = lax.axis_index("c")
        s = lax.axis_index("s")
        w = c * NUM_TILES + s
        pltpu.sync_copy(dst_hbm.at[w], dst_v)
        _zero_fill(ones_v, CHUNK, _CNT_W)
        zbase = s * STRIPE
        for off in range(0, STRIPE, CHUNK):
            nrow = min(CHUNK, STRIPE - off)
            pltpu.sync_copy(ones_v.at[pl.ds(0, nrow)],
                            acc.at[pl.ds(zbase + off, nrow)])

        @pl.loop(0, CHUNK)
        def _(i):
            ones_v[i, pl.ds(0, _CNT_W)] = jnp.ones((_CNT_W,), jnp.float32)

        plsc.subcore_barrier()

        @pl.loop(0, CHUNKS_PER_TILE, step=_CNT_NBUF)
        def _(j):
            sds = [pltpu.async_copy(ones_v, acc.at[dst_v.at[j + b]],
                                    sem, add=True)
                   for b in range(_CNT_NBUF)]
            for d in sds:
                d.wait()

        plsc.subcore_barrier()
        _copy_out(acc, out0, out1, c, s)

    return cnt(dst_hbm_arr)


_BLK = 2000


def _tc_layer1(a0a, a1a, a0b, a1b, x, c0, c1, wl, wr, b):
    def body(a0a_ref, a1a_ref, a0b_ref, a1b_ref, x_ref, c0_ref, c1_ref,
             wl_ref, wr_ref, b_ref, h_ref):
        cntv = jnp.maximum(c0_ref[:, 0:1] + c1_ref[:, 0:1], 1.0)
        agg = jnp.concatenate([a0a_ref[...] + a1a_ref[...],
                               a0b_ref[...] + a1b_ref[...]], axis=1)
        mean1 = agg / cntv
        h = (lax.dot_general(mean1, wl_ref[...], _MM) + b_ref[...]
             + lax.dot_general(x_ref[...], wr_ref[...], _MM))
        h_ref[...] = jnp.maximum(h, 0.0)

    return pl.pallas_call(
        body,
        grid=(N_NODES // _BLK,),
        in_specs=[
            pl.BlockSpec((_BLK, HID), lambda i: (i, 0)),
            pl.BlockSpec((_BLK, HID), lambda i: (i, 0)),
            pl.BlockSpec((_BLK, HID), lambda i: (i, 0)),
            pl.BlockSpec((_BLK, HID), lambda i: (i, 0)),
            pl.BlockSpec((_BLK, IN_CH), lambda i: (i, 0)),
            pl.BlockSpec((_BLK, _CNT_W), lambda i: (i, 0)),
            pl.BlockSpec((_BLK, _CNT_W), lambda i: (i, 0)),
            pl.BlockSpec((IN_CH, HID), lambda i: (0, 0)),
            pl.BlockSpec((IN_CH, HID), lambda i: (0, 0)),
            pl.BlockSpec((1, HID), lambda i: (0, 0)),
        ],
        out_specs=pl.BlockSpec((_BLK, HID), lambda i: (i, 0)),
        out_shape=jax.ShapeDtypeStruct((N_NODES, HID), jnp.float32),
    )(a0a, a1a, a0b, a1b, x, c0, c1, wl, wr, b)


def _tc_layer2(b0, b1, h1, c0, c1, wl, wr, b, wc, bc):
    def body(b0_ref, b1_ref, h1_ref, c0_ref, c1_ref, wl_ref, wr_ref, b_ref,
             wc_ref, bc_ref, o_ref):
        cntv = jnp.maximum(c0_ref[:, 0:1] + c1_ref[:, 0:1], 1.0)
        mean2 = (b0_ref[...] + b1_ref[...]) / cntv
        h2 = (lax.dot_general(mean2, wl_ref[...], _MM) + b_ref[...]
              + lax.dot_general(h1_ref[...], wr_ref[...], _MM))
        h2 = jnp.maximum(h2, 0.0)
        o_ref[...] = lax.dot_general(h2, wc_ref[...], _MM) + bc_ref[...]

    return pl.pallas_call(
        body,
        grid=(N_NODES // _BLK,),
        in_specs=[
            pl.BlockSpec((_BLK, HID), lambda i: (i, 0)),
            pl.BlockSpec((_BLK, HID), lambda i: (i, 0)),
            pl.BlockSpec((_BLK, HID), lambda i: (i, 0)),
            pl.BlockSpec((_BLK, _CNT_W), lambda i: (i, 0)),
            pl.BlockSpec((_BLK, _CNT_W), lambda i: (i, 0)),
            pl.BlockSpec((HID, HID2), lambda i: (0, 0)),
            pl.BlockSpec((HID, HID2), lambda i: (0, 0)),
            pl.BlockSpec((1, HID2), lambda i: (0, 0)),
            pl.BlockSpec((HID2, OUT_CH), lambda i: (0, 0)),
            pl.BlockSpec((1, OUT_CH), lambda i: (0, 0)),
        ],
        out_specs=pl.BlockSpec((_BLK, OUT_CH), lambda i: (i, 0)),
        out_shape=jax.ShapeDtypeStruct((N_NODES, OUT_CH), jnp.float32),
    )(b0, b1, h1, c0, c1, wl, wr, b, wc, bc)


def kernel(x, edge_index, Wl1, bl1, Wr1, Wl2, bl2, Wr2, Wc, bc):
    ei = edge_index.astype(jnp.int32)
    pad = E_PAD - N_EDGES
    src = jnp.concatenate([ei[0], jnp.zeros((pad,), jnp.int32)])
    dst = jnp.concatenate([ei[1], jnp.full((pad,), DUMMY_ROW, jnp.int32)])
    src = src.reshape(TOT_CHUNKS, CHUNK)
    dst = dst.reshape(TOT_CHUNKS, CHUNK)
    n0 = NUM_TILES * CH_C0
    src0 = src[:n0].reshape(NUM_TILES, CH_C0, CHUNK)
    dst0 = dst[:n0].reshape(NUM_TILES, CH_C0, CHUNK)
    src1 = src[n0:].reshape(NUM_TILES, CH_C1, CHUNK)
    dst1 = dst[n0:].reshape(NUM_TILES, CH_C1, CHUNK)
    idx = (src0, dst0, src1, dst1)

    cnt0, cnt1 = _sc_count(dst.reshape(NW, CHUNKS_PER_TILE, CHUNK))
    agg64 = _make_sc_agg(HID, 4, CH_C0, CH_C1)
    a0a, a1a = agg64(x[:, :HID], *idx)
    a0b, a1b = agg64(x[:, HID:], *idx)
    h1 = _tc_layer1(a0a, a1a, a0b, a1b, x, cnt0, cnt1, Wl1.T, Wr1.T,
                    bl1.reshape(1, HID))
    b0, b1 = agg64(h1, *idx)
    return _tc_layer2(b0, b1, h1, cnt0, cnt1, Wl2.T, Wr2.T,
                      bl2.reshape(1, HID2), Wc.T, bc.reshape(1, OUT_CH))


# speedup vs baseline: 2.4355x; 1.9563x over previous
import functools

import jax
import jax.numpy as jnp
from jax import lax
from jax.experimental import pallas as pl
from jax.experimental.pallas import tpu as pltpu
from jax.experimental.pallas import tpu_sc as plsc

N_NODES = 10000
N_EDGES = 320000
IN_CH = 128
HID = 64
HID2 = 32
OUT_CH = 1

NUM_SC = 2
NUM_TILES = 16
NW = NUM_SC * NUM_TILES
CHUNK = 128
CHUNKS_PER_TILE = 80
EDGES_PER_TILE = CHUNK * CHUNKS_PER_TILE
E_PAD = EDGES_PER_TILE * NW
TOT_CHUNKS = E_PAD // CHUNK
CH_ALL = TOT_CHUNKS // NUM_TILES
DUMMY_ROW = N_NODES
ACC_ROWS = 10112
STRIPE = ACC_ROWS // NUM_TILES
LAST_STRIPE = N_NODES - (NUM_TILES - 1) * STRIPE

_MESH = plsc.VectorSubcoreMesh(core_axis_name="c", subcore_axis_name="s")
_SC_PARAMS = pltpu.CompilerParams(use_tc_tiling_on_sc=False)
_MM = (((1,), (0,)), ((), ()))


def _zero_fill(buf, nrows, width):
    @pl.loop(0, nrows)
    def _(i):
        @pl.loop(0, width // 16)
        def _(k):
            buf[i, pl.ds(k * 16, 16)] = jnp.zeros((16,), jnp.float32)


def _make_sc_agg(width, nbuf):
    out_t = [jax.ShapeDtypeStruct((N_NODES, width), jnp.float32)] * 2
    assert CH_ALL % nbuf == 0

    @functools.partial(
        pl.kernel,
        out_type=out_t,
        mesh=_MESH,
        compiler_params=_SC_PARAMS,
        scratch_types=[
            pltpu.VMEM((CH_ALL, CHUNK), jnp.int32),
            pltpu.VMEM((CH_ALL, CHUNK), jnp.int32),
            pltpu.VMEM((nbuf, CHUNK, width), jnp.float32),
            pltpu.VMEM_SHARED((ACC_ROWS, width), jnp.float32),
            pltpu.VMEM_SHARED((ACC_ROWS, width), jnp.float32),
            pltpu.SemaphoreType.DMA((nbuf,)),
            pltpu.SemaphoreType.DMA((nbuf,)),
        ],
    )
    def agg(tbl0_hbm, tbl1_hbm, src_hbm, dst_hbm, out0, out1,
            src_v, dst_v, rows_v, acc, tbl, gsem, ssem):
        c = lax.axis_index("c")
        s = lax.axis_index("s")
        zbase = s * STRIPE

        def stage(tbl_h):
            @pl.when(s < NUM_TILES - 1)
            def _():
                pltpu.sync_copy(tbl_h.at[pl.ds(zbase, STRIPE)],
                                tbl.at[pl.ds(zbase, STRIPE)])

            @pl.when(s == NUM_TILES - 1)
            def _():
                pltpu.sync_copy(tbl_h.at[pl.ds(zbase, LAST_STRIPE)],
                                tbl.at[pl.ds(zbase, LAST_STRIPE)])

        @pl.when(c == 0)
        def _():
            stage(tbl0_hbm)

        @pl.when(c == 1)
        def _():
            stage(tbl1_hbm)

        pltpu.sync_copy(src_hbm.at[s], src_v)
        pltpu.sync_copy(dst_hbm.at[s], dst_v)
        _zero_fill(rows_v.at[0], CHUNK, width)
        for off in range(0, STRIPE, CHUNK):
            nrow = min(CHUNK, STRIPE - off)
            pltpu.sync_copy(rows_v.at[0].at[pl.ds(0, nrow)],
                            acc.at[pl.ds(zbase + off, nrow)])
        plsc.subcore_barrier()

        @pl.loop(0, CH_ALL, step=nbuf)
        def _(j):
            gds = [pltpu.async_copy(tbl.at[src_v.at[j + b]],
                                    rows_v.at[b], gsem.at[b])
                   for b in range(nbuf)]
            sds = []
            for b in range(nbuf):
                gds[b].wait()
                sds.append(pltpu.async_copy(rows_v.at[b],
                                            acc.at[dst_v.at[j + b]],
                                            ssem.at[b], add=True))
            for b in range(nbuf):
                sds[b].wait()

        plsc.subcore_barrier()
        _copy_out(acc, out0, out1, c, s)

    return agg


def _copy_out(acc, out0, out1, c, s):
    ob = s * STRIPE

    def stripe_to(dst):
        @pl.when(s < NUM_TILES - 1)
        def _():
            pltpu.sync_copy(acc.at[pl.ds(ob, STRIPE)],
                            dst.at[pl.ds(ob, STRIPE)])

        @pl.when(s == NUM_TILES - 1)
        def _():
            pltpu.sync_copy(acc.at[pl.ds(ob, LAST_STRIPE)],
                            dst.at[pl.ds(ob, LAST_STRIPE)])

    @pl.when(c == 0)
    def _():
        stripe_to(out0)

    @pl.when(c == 1)
    def _():
        stripe_to(out1)


_CNT_W = 16
_CNT_NBUF = 8


def _sc_count(dst_hbm_arr):
    out_t = [jax.ShapeDtypeStruct((N_NODES, _CNT_W), jnp.float32)] * 2

    @functools.partial(
        pl.kernel,
        out_type=out_t,
        mesh=_MESH,
        compiler_params=_SC_PARAMS,
        scratch_types=[
            pltpu.VMEM((CHUNKS_PER_TILE, CHUNK), jnp.int32),
            pltpu.VMEM((CHUNK, _CNT_W), jnp.float32),
            pltpu.VMEM_SHARED((ACC_ROWS, _CNT_W), jnp.float32),
            pltpu.SemaphoreType.DMA,
        ],
    )
    def cnt(dst_hbm, out0, out1, dst_v, ones_v, acc, sem):
        c = lax.axis_index("c")
        s = lax.axis_index("s")
        w = c * NUM_TILES + s
        pltpu.sync_copy(dst_hbm.at[w], dst_v)
        _zero_fill(ones_v, CHUNK, _CNT_W)
        zbase = s * STRIPE
        for off in range(0, STRIPE, CHUNK):
            nrow = min(CHUNK, STRIPE - off)
            pltpu.sync_copy(ones_v.at[pl.ds(0, nrow)],
                            acc.at[pl.ds(zbase + off, nrow)])

        @pl.loop(0, CHUNK)
        def _(i):
            ones_v[i, pl.ds(0, _CNT_W)] = jnp.ones((_CNT_W,), jnp.float32)

        plsc.subcore_barrier()

        @pl.loop(0, CHUNKS_PER_TILE, step=_CNT_NBUF)
        def _(j):
            sds = [pltpu.async_copy(ones_v, acc.at[dst_v.at[j + b]],
                                    sem, add=True)
                   for b in range(_CNT_NBUF)]
            for d in sds:
                d.wait()

        plsc.subcore_barrier()
        _copy_out(acc, out0, out1, c, s)

    return cnt(dst_hbm_arr)


_BLK = 2000


def _tc_layer1(aA0, aA1, aB0, aB1, x, c0, c1, wl, wr, b):
    def body(aA0_ref, aA1_ref, aB0_ref, aB1_ref, x_ref, c0_ref, c1_ref,
             wl_ref, wr_ref, b_ref, h_ref):
        cntv = jnp.maximum(c0_ref[:, 0:1] + c1_ref[:, 0:1], 1.0)
        agg = jnp.concatenate([aA0_ref[...], aA1_ref[...],
                               aB0_ref[...], aB1_ref[...]], axis=1)
        mean1 = agg / cntv
        h = (lax.dot_general(mean1, wl_ref[...], _MM) + b_ref[...]
             + lax.dot_general(x_ref[...], wr_ref[...], _MM))
        h_ref[...] = jnp.maximum(h, 0.0)

    return pl.pallas_call(
        body,
        grid=(N_NODES // _BLK,),
        in_specs=[
            pl.BlockSpec((_BLK, HID2), lambda i: (i, 0)),
            pl.BlockSpec((_BLK, HID2), lambda i: (i, 0)),
            pl.BlockSpec((_BLK, HID2), lambda i: (i, 0)),
            pl.BlockSpec((_BLK, HID2), lambda i: (i, 0)),
            pl.BlockSpec((_BLK, IN_CH), lambda i: (i, 0)),
            pl.BlockSpec((_BLK, _CNT_W), lambda i: (i, 0)),
            pl.BlockSpec((_BLK, _CNT_W), lambda i: (i, 0)),
            pl.BlockSpec((IN_CH, HID), lambda i: (0, 0)),
            pl.BlockSpec((IN_CH, HID), lambda i: (0, 0)),
            pl.BlockSpec((1, HID), lambda i: (0, 0)),
        ],
        out_specs=pl.BlockSpec((_BLK, HID), lambda i: (i, 0)),
        out_shape=jax.ShapeDtypeStruct((N_NODES, HID), jnp.float32),
    )(aA0, aA1, aB0, aB1, x, c0, c1, wl, wr, b)


def _tc_layer2(b0, b1, h1, c0, c1, wl, wr, b, wc, bc):
    def body(b0_ref, b1_ref, h1_ref, c0_ref, c1_ref, wl_ref, wr_ref, b_ref,
             wc_ref, bc_ref, o_ref):
        cntv = jnp.maximum(c0_ref[:, 0:1] + c1_ref[:, 0:1], 1.0)
        mean2 = jnp.concatenate([b0_ref[...], b1_ref[...]], axis=1) / cntv
        h2 = (lax.dot_general(mean2, wl_ref[...], _MM) + b_ref[...]
              + lax.dot_general(h1_ref[...], wr_ref[...], _MM))
        h2 = jnp.maximum(h2, 0.0)
        o_ref[...] = lax.dot_general(h2, wc_ref[...], _MM) + bc_ref[...]

    return pl.pallas_call(
        body,
        grid=(N_NODES // _BLK,),
        in_specs=[
            pl.BlockSpec((_BLK, HID2), lambda i: (i, 0)),
            pl.BlockSpec((_BLK, HID2), lambda i: (i, 0)),
            pl.BlockSpec((_BLK, HID), lambda i: (i, 0)),
            pl.BlockSpec((_BLK, _CNT_W), lambda i: (i, 0)),
            pl.BlockSpec((_BLK, _CNT_W), lambda i: (i, 0)),
            pl.BlockSpec((HID, HID2), lambda i: (0, 0)),
            pl.BlockSpec((HID, HID2), lambda i: (0, 0)),
            pl.BlockSpec((1, HID2), lambda i: (0, 0)),
            pl.BlockSpec((HID2, OUT_CH), lambda i: (0, 0)),
            pl.BlockSpec((1, OUT_CH), lambda i: (0, 0)),
        ],
        out_specs=pl.BlockSpec((_BLK, OUT_CH), lambda i: (i, 0)),
        out_shape=jax.ShapeDtypeStruct((N_NODES, OUT_CH), jnp.float32),
    )(b0, b1, h1, c0, c1, wl, wr, b, wc, bc)


def kernel(x, edge_index, Wl1, bl1, Wr1, Wl2, bl2, Wr2, Wc, bc):
    ei = edge_index.astype(jnp.int32)
    pad = E_PAD - N_EDGES
    src = jnp.concatenate([ei[0], jnp.zeros((pad,), jnp.int32)])
    dst = jnp.concatenate([ei[1], jnp.full((pad,), DUMMY_ROW, jnp.int32)])
    src = src.reshape(TOT_CHUNKS, CHUNK)
    dst = dst.reshape(TOT_CHUNKS, CHUNK)
    srcT = src.reshape(NUM_TILES, CH_ALL, CHUNK)
    dstT = dst.reshape(NUM_TILES, CH_ALL, CHUNK)

    cnt0, cnt1 = _sc_count(dst.reshape(NW, CHUNKS_PER_TILE, CHUNK))
    agg32 = _make_sc_agg(HID2, 8)
    aA0, aA1 = agg32(x[:, 0:32], x[:, 32:64], srcT, dstT)
    aB0, aB1 = agg32(x[:, 64:96], x[:, 96:128], srcT, dstT)
    h1 = _tc_layer1(aA0, aA1, aB0, aB1, x, cnt0, cnt1, Wl1.T, Wr1.T,
                    bl1.reshape(1, HID))
    b0, b1 = agg32(h1[:, 0:32], h1[:, 32:64], srcT, dstT)
    return _tc_layer2(b0, b1, h1, cnt0, cnt1, Wl2.T, Wr2.T,
                      bl2.reshape(1, HID2), Wc.T, bc.reshape(1, OUT_CH))


# speedup vs baseline: 2.4806x; 1.0186x over previous
import functools

import jax
import jax.numpy as jnp
from jax import lax
from jax.experimental import pallas as pl
from jax.experimental.pallas import tpu as pltpu
from jax.experimental.pallas import tpu_sc as plsc

N_NODES = 10000
N_EDGES = 320000
IN_CH = 128
HID = 64
HID2 = 32
OUT_CH = 1

NUM_SC = 2
NUM_TILES = 16
NW = NUM_SC * NUM_TILES
CHUNK = 128
CHUNKS_PER_TILE = 80
EDGES_PER_TILE = CHUNK * CHUNKS_PER_TILE
E_PAD = EDGES_PER_TILE * NW
TOT_CHUNKS = E_PAD // CHUNK
CH_ALL = TOT_CHUNKS // NUM_TILES
DUMMY_ROW = N_NODES
ACC_ROWS = 10112
STRIPE = ACC_ROWS // NUM_TILES
LAST_STRIPE = N_NODES - (NUM_TILES - 1) * STRIPE

_MESH = plsc.VectorSubcoreMesh(core_axis_name="c", subcore_axis_name="s")
_SC_PARAMS = pltpu.CompilerParams(use_tc_tiling_on_sc=False)
_MM = (((1,), (0,)), ((), ()))


def _zero_fill(buf, nrows, width):
    @pl.loop(0, nrows)
    def _(i):
        @pl.loop(0, width // 16)
        def _(k):
            buf[i, pl.ds(k * 16, 16)] = jnp.zeros((16,), jnp.float32)


def _make_sc_agg(width, nbuf):
    out_t = [jax.ShapeDtypeStruct((N_NODES, width), jnp.float32)] * 2
    assert CH_ALL % nbuf == 0

    @functools.partial(
        pl.kernel,
        out_type=out_t,
        mesh=_MESH,
        compiler_params=_SC_PARAMS,
        scratch_types=[
            pltpu.VMEM((CH_ALL, CHUNK), jnp.int32),
            pltpu.VMEM((CH_ALL, CHUNK), jnp.int32),
            pltpu.VMEM((nbuf, CHUNK, width), jnp.float32),
            pltpu.VMEM_SHARED((ACC_ROWS, width), jnp.float32),
            pltpu.VMEM_SHARED((ACC_ROWS, width), jnp.float32),
            pltpu.SemaphoreType.DMA((nbuf,)),
            pltpu.SemaphoreType.DMA((nbuf,)),
        ],
    )
    def agg(tbl0_hbm, tbl1_hbm, src_hbm, dst_hbm, out0, out1,
            src_v, dst_v, rows_v, acc, tbl, gsem, ssem):
        c = lax.axis_index("c")
        s = lax.axis_index("s")
        zbase = s * STRIPE

        def stage(tbl_h):
            @pl.when(s < NUM_TILES - 1)
            def _():
                pltpu.sync_copy(tbl_h.at[pl.ds(zbase, STRIPE)],
                                tbl.at[pl.ds(zbase, STRIPE)])

            @pl.when(s == NUM_TILES - 1)
            def _():
                pltpu.sync_copy(tbl_h.at[pl.ds(zbase, LAST_STRIPE)],
                                tbl.at[pl.ds(zbase, LAST_STRIPE)])

        @pl.when(c == 0)
        def _():
            stage(tbl0_hbm)

        @pl.when(c == 1)
        def _():
            stage(tbl1_hbm)

        pltpu.sync_copy(src_hbm.at[s], src_v)
        pltpu.sync_copy(dst_hbm.at[s], dst_v)
        _zero_fill(rows_v.at[0], CHUNK, width)
        for off in range(0, STRIPE, CHUNK):
            nrow = min(CHUNK, STRIPE - off)
            pltpu.sync_copy(rows_v.at[0].at[pl.ds(0, nrow)],
                            acc.at[pl.ds(zbase + off, nrow)])
        plsc.subcore_barrier()

        @pl.loop(0, CH_ALL, step=nbuf)
        def _(j):
            gds = [pltpu.async_copy(tbl.at[src_v.at[j + b]],
                                    rows_v.at[b], gsem.at[b])
                   for b in range(nbuf)]
            for b in range(nbuf):
                gds[b].wait()
                pltpu.sync_copy(rows_v.at[b], acc.at[dst_v.at[j + b]],
                                add=True)

        plsc.subcore_barrier()
        _copy_out(acc, out0, out1, c, s)

    return agg


def _copy_out(acc, out0, out1, c, s):
    ob = s * STRIPE

    def stripe_to(dst):
        @pl.when(s < NUM_TILES - 1)
        def _():
            pltpu.sync_copy(acc.at[pl.ds(ob, STRIPE)],
                            dst.at[pl.ds(ob, STRIPE)])

        @pl.when(s == NUM_TILES - 1)
        def _():
            pltpu.sync_copy(acc.at[pl.ds(ob, LAST_STRIPE)],
                            dst.at[pl.ds(ob, LAST_STRIPE)])

    @pl.when(c == 0)
    def _():
        stripe_to(out0)

    @pl.when(c == 1)
    def _():
        stripe_to(out1)


_CNT_W = 16
_CNT_NBUF = 8


def _sc_count(dst_hbm_arr):
    out_t = [jax.ShapeDtypeStruct((N_NODES, _CNT_W), jnp.float32)] * 2

    @functools.partial(
        pl.kernel,
        out_type=out_t,
        mesh=_MESH,
        compiler_params=_SC_PARAMS,
        scratch_types=[
            pltpu.VMEM((CHUNKS_PER_TILE, CHUNK), jnp.int32),
            pltpu.VMEM((CHUNK, _CNT_W), jnp.float32),
            pltpu.VMEM_SHARED((ACC_ROWS, _CNT_W), jnp.float32),
            pltpu.SemaphoreType.DMA,
        ],
    )
    def cnt(dst_hbm, out0, out1, dst_v, ones_v, acc, sem):
        c = lax.axis_index("c")
        s = lax.axis_index("s")
        w = c * NUM_TILES + s
        pltpu.sync_copy(dst_hbm.at[w], dst_v)
        _zero_fill(ones_v, CHUNK, _CNT_W)
        zbase = s * STRIPE
        for off in range(0, STRIPE, CHUNK):
            nrow = min(CHUNK, STRIPE - off)
            pltpu.sync_copy(ones_v.at[pl.ds(0, nrow)],
                            acc.at[pl.ds(zbase + off, nrow)])

        @pl.loop(0, CHUNK)
        def _(i):
            ones_v[i, pl.ds(0, _CNT_W)] = jnp.ones((_CNT_W,), jnp.float32)

        plsc.subcore_barrier()

        @pl.loop(0, CHUNKS_PER_TILE)
        def _(j):
            pltpu.sync_copy(ones_v, acc.at[dst_v.at[j]], add=True)

        plsc.subcore_barrier()
        _copy_out(acc, out0, out1, c, s)

    return cnt(dst_hbm_arr)


_BLK = 2000


def _tc_layer1(aA0, aA1, aB0, aB1, x, c0, c1, wl, wr, b):
    def body(aA0_ref, aA1_ref, aB0_ref, aB1_ref, x_ref, c0_ref, c1_ref,
             wl_ref, wr_ref, b_ref, h_ref):
        cntv = jnp.maximum(c0_ref[:, 0:1] + c1_ref[:, 0:1], 1.0)
        agg = jnp.concatenate([aA0_ref[...], aA1_ref[...],
                               aB0_ref[...], aB1_ref[...]], axis=1)
        mean1 = agg / cntv
        h = (lax.dot_general(mean1, wl_ref[...], _MM) + b_ref[...]
             + lax.dot_general(x_ref[...], wr_ref[...], _MM))
        h_ref[...] = jnp.maximum(h, 0.0)

    return pl.pallas_call(
        body,
        grid=(N_NODES // _BLK,),
        in_specs=[
            pl.BlockSpec((_BLK, HID2), lambda i: (i, 0)),
            pl.BlockSpec((_BLK, HID2), lambda i: (i, 0)),
            pl.BlockSpec((_BLK, HID2), lambda i: (i, 0)),
            pl.BlockSpec((_BLK, HID2), lambda i: (i, 0)),
            pl.BlockSpec((_BLK, IN_CH), lambda i: (i, 0)),
            pl.BlockSpec((_BLK, _CNT_W), lambda i: (i, 0)),
            pl.BlockSpec((_BLK, _CNT_W), lambda i: (i, 0)),
            pl.BlockSpec((IN_CH, HID), lambda i: (0, 0)),
            pl.BlockSpec((IN_CH, HID), lambda i: (0, 0)),
            pl.BlockSpec((1, HID), lambda i: (0, 0)),
        ],
        out_specs=pl.BlockSpec((_BLK, HID), lambda i: (i, 0)),
        out_shape=jax.ShapeDtypeStruct((N_NODES, HID), jnp.float32),
    )(aA0, aA1, aB0, aB1, x, c0, c1, wl, wr, b)


def _tc_layer2(b0, b1, h1, c0, c1, wl, wr, b, wc, bc):
    def body(b0_ref, b1_ref, h1_ref, c0_ref, c1_ref, wl_ref, wr_ref, b_ref,
             wc_ref, bc_ref, o_ref):
        cntv = jnp.maximum(c0_ref[:, 0:1] + c1_ref[:, 0:1], 1.0)
        mean2 = jnp.concatenate([b0_ref[...], b1_ref[...]], axis=1) / cntv
        h2 = (lax.dot_general(mean2, wl_ref[...], _MM) + b_ref[...]
              + lax.dot_general(h1_ref[...], wr_ref[...], _MM))
        h2 = jnp.maximum(h2, 0.0)
        o_ref[...] = lax.dot_general(h2, wc_ref[...], _MM) + bc_ref[...]

    return pl.pallas_call(
        body,
        grid=(N_NODES // _BLK,),
        in_specs=[
            pl.BlockSpec((_BLK, HID2), lambda i: (i, 0)),
            pl.BlockSpec((_BLK, HID2), lambda i: (i, 0)),
            pl.BlockSpec((_BLK, HID), lambda i: (i, 0)),
            pl.BlockSpec((_BLK, _CNT_W), lambda i: (i, 0)),
            pl.BlockSpec((_BLK, _CNT_W), lambda i: (i, 0)),
            pl.BlockSpec((HID, HID2), lambda i: (0, 0)),
            pl.BlockSpec((HID, HID2), lambda i: (0, 0)),
            pl.BlockSpec((1, HID2), lambda i: (0, 0)),
            pl.BlockSpec((HID2, OUT_CH), lambda i: (0, 0)),
            pl.BlockSpec((1, OUT_CH), lambda i: (0, 0)),
        ],
        out_specs=pl.BlockSpec((_BLK, OUT_CH), lambda i: (i, 0)),
        out_shape=jax.ShapeDtypeStruct((N_NODES, OUT_CH), jnp.float32),
    )(b0, b1, h1, c0, c1, wl, wr, b, wc, bc)


def kernel(x, edge_index, Wl1, bl1, Wr1, Wl2, bl2, Wr2, Wc, bc):
    ei = edge_index.astype(jnp.int32)
    pad = E_PAD - N_EDGES
    src = jnp.concatenate([ei[0], jnp.zeros((pad,), jnp.int32)])
    dst = jnp.concatenate([ei[1], jnp.full((pad,), DUMMY_ROW, jnp.int32)])
    src = src.reshape(TOT_CHUNKS, CHUNK)
    dst = dst.reshape(TOT_CHUNKS, CHUNK)
    srcT = src.reshape(NUM_TILES, CH_ALL, CHUNK)
    dstT = dst.reshape(NUM_TILES, CH_ALL, CHUNK)

    cnt0, cnt1 = _sc_count(dst.reshape(NW, CHUNKS_PER_TILE, CHUNK))
    agg32 = _make_sc_agg(HID2, 8)
    aA0, aA1 = agg32(x[:, 0:32], x[:, 32:64], srcT, dstT)
    aB0, aB1 = agg32(x[:, 64:96], x[:, 96:128], srcT, dstT)
    h1 = _tc_layer1(aA0, aA1, aB0, aB1, x, cnt0, cnt1, Wl1.T, Wr1.T,
                    bl1.reshape(1, HID))
    b0, b1 = agg32(h1[:, 0:32], h1[:, 32:64], srcT, dstT)
    return _tc_layer2(b0, b1, h1, cnt0, cnt1, Wl2.T, Wr2.T,
                      bl2.reshape(1, HID2), Wc.T, bc.reshape(1, OUT_CH))
